# Initial kernel scaffold; baseline (speedup 1.0000x reference)
import jax
import jax.numpy as jnp
from jax.experimental import pallas as pl


def kernel(x, qweight, absmax, code):
    raise NotImplementedError("write your pallas kernel here")


# speedup vs baseline: 385.5316x; 385.5316x over previous
import functools

import jax
import jax.numpy as jnp
from jax import lax
from jax.experimental import pallas as pl
from jax.experimental.pallas import tpu as pltpu
from jax.experimental.pallas import tpu_sc as plsc

VOCAB = 100000
DIM = 128
BLOCK = 4096
N_BLOCKS = (VOCAB * DIM) // BLOCK
ROWS_PER_ABSMAX = BLOCK // DIM
BATCH = 4096
FIELDS = 26

NC, NS, L = 2, 16, 16
NW = NC * NS


def _build(n_rows, chunk, vocab, n_blocks):
    bpw = n_rows // NW
    nchunk = bpw // chunk
    mesh = plsc.VectorSubcoreMesh(core_axis_name="c", subcore_axis_name="s")

    @functools.partial(
        pl.kernel,
        out_type=jax.ShapeDtypeStruct((n_rows, DIM), jnp.float32),
        mesh=mesh,
        compiler_params=pltpu.CompilerParams(
            use_tc_tiling_on_sc=False, needs_layout_passes=False),
        scratch_types=[
            pltpu.VMEM((nchunk, chunk), jnp.int32),
            pltpu.VMEM((n_blocks,), jnp.float32),
            pltpu.VMEM((256,), jnp.float32),
            pltpu.VMEM((chunk, DIM), jnp.int32),
            pltpu.VMEM((chunk, DIM), jnp.float32),
            pltpu.VMEM((2 * L,), jnp.float32),
            pltpu.SemaphoreType.DMA,
        ],
    )
    def k(x_hbm, qw_hbm, absmax_hbm, code_hbm, out_hbm,
          idx_v, absmax_v, code_v, rows_v, out_v, scales_v, sem):
        wid = lax.axis_index("s") * NC + lax.axis_index("c")
        pltpu.sync_copy(x_hbm.at[wid], idx_v)
        pltpu.sync_copy(absmax_hbm, absmax_v)
        pltpu.sync_copy(code_hbm, code_v)

        def chunk_body(c, carry):
            pltpu.async_copy(qw_hbm.at[idx_v.at[c]], rows_v, sem).wait()

            def group_body(g, carry2):
                idx16 = idx_v[c, pl.ds(g * L, L)]
                shift5 = jnp.full((L,), 5, jnp.int32)
                scales_v[pl.ds(L, L)] = plsc.load_gather(
                    absmax_v, [lax.shift_right_logical(idx16, shift5)])
                for j in range(L):
                    scale = plsc.load_gather(
                        scales_v, [jnp.full((L,), L + j, jnp.int32)])
                    r = g * L + j
                    for kk in range(DIM // L):
                        q = rows_v[r, pl.ds(kk * L, L)]
                        out_v[r, pl.ds(kk * L, L)] = (
                            plsc.load_gather(code_v, [q]) * scale)
                return carry2

            lax.fori_loop(0, chunk // L, group_body, 0, unroll=False)
            pltpu.sync_copy(
                out_v, out_hbm.at[pl.ds(wid * bpw + c * chunk, chunk)])
            return carry

        lax.fori_loop(0, nchunk, chunk_body, 0, unroll=False)

    return k


_KERNEL = _build(BATCH * FIELDS, 128, VOCAB, N_BLOCKS)


def kernel(x, qweight, absmax, code):
    n_rows = BATCH * FIELDS
    bpw = n_rows // NW
    xr = x.reshape(NW, bpw // 128, 128)
    out = _KERNEL(xr, qweight, absmax, code)
    return out.reshape(BATCH, FIELDS, DIM)


# speedup vs baseline: 453.0044x; 1.1750x over previous
import functools

import jax
import jax.numpy as jnp
from jax import lax
from jax.experimental import pallas as pl
from jax.experimental.pallas import tpu as pltpu
from jax.experimental.pallas import tpu_sc as plsc

VOCAB = 100000
DIM = 128
BLOCK = 4096
N_BLOCKS = (VOCAB * DIM) // BLOCK
ROWS_PER_ABSMAX = BLOCK // DIM
BATCH = 4096
FIELDS = 26

NC, NS, L = 2, 16, 16
NW = NC * NS


def _build(n_rows, chunk, vocab, n_blocks):
    bpw = n_rows // NW
    nchunk = bpw // chunk
    mesh = plsc.VectorSubcoreMesh(core_axis_name="c", subcore_axis_name="s")

    @functools.partial(
        pl.kernel,
        out_type=jax.ShapeDtypeStruct((n_rows, DIM), jnp.float32),
        mesh=mesh,
        compiler_params=pltpu.CompilerParams(
            use_tc_tiling_on_sc=False, needs_layout_passes=False),
        scratch_types=[
            pltpu.VMEM((nchunk, chunk), jnp.int32),
            pltpu.VMEM((n_blocks,), jnp.float32),
            pltpu.VMEM((256,), jnp.float32),
            pltpu.VMEM((2, chunk, DIM), jnp.int32),
            pltpu.VMEM((2, chunk, DIM), jnp.float32),
            pltpu.VMEM((2 * L,), jnp.float32),
            pltpu.SemaphoreType.DMA,
            pltpu.SemaphoreType.DMA,
            pltpu.SemaphoreType.DMA,
            pltpu.SemaphoreType.DMA,
        ],
    )
    def k(x_hbm, qw_hbm, absmax_hbm, code_hbm, out_hbm,
          idx_v, absmax_v, code_v, rows_v, out_v, scales_v,
          in_sem0, in_sem1, out_sem0, out_sem1):
        in_sems = [in_sem0, in_sem1]
        out_sems = [out_sem0, out_sem1]
        wid = lax.axis_index("s") * NC + lax.axis_index("c")
        pltpu.sync_copy(x_hbm.at[wid], idx_v)
        pltpu.sync_copy(absmax_hbm, absmax_v)
        pltpu.sync_copy(code_hbm, code_v)

        for b in range(2):
            pltpu.async_copy(qw_hbm.at[idx_v.at[b]], rows_v.at[b], in_sems[b])

        def dequant_chunk(c, b):
            def group_body(g, carry2):
                idx16 = idx_v[c, pl.ds(g * L, L)]
                shift5 = jnp.full((L,), 5, jnp.int32)
                scales_v[pl.ds(L, L)] = plsc.load_gather(
                    absmax_v, [lax.shift_right_logical(idx16, shift5)])
                for j in range(L):
                    scale = plsc.load_gather(
                        scales_v, [jnp.full((L,), L + j, jnp.int32)])
                    r = g * L + j
                    for kk in range(DIM // L):
                        q = rows_v[b, r, pl.ds(kk * L, L)]
                        out_v[b, r, pl.ds(kk * L, L)] = (
                            plsc.load_gather(code_v, [q]) * scale)
                return carry2

            lax.fori_loop(0, chunk // L, group_body, 0, unroll=False)

        def pair_body(g, carry):
            for b in range(2):
                c = g * 2 + b
                pltpu.make_async_copy(
                    qw_hbm.at[pl.ds(0, chunk)], rows_v.at[b],
                    in_sems[b]).wait()
                @pl.when(c >= 2)
                def _():
                    pltpu.make_async_copy(
                        out_v.at[b], out_hbm.at[pl.ds(0, chunk)],
                        out_sems[b]).wait()

                dequant_chunk(c, b)

                pltpu.async_copy(
                    out_v.at[b],
                    out_hbm.at[pl.ds(wid * bpw + c * chunk, chunk)],
                    out_sems[b])
                @pl.when(c + 2 < nchunk)
                def _():
                    pltpu.async_copy(
                        qw_hbm.at[idx_v.at[c + 2]], rows_v.at[b], in_sems[b])
            return carry

        lax.fori_loop(0, nchunk // 2, pair_body, 0, unroll=False)

        for b in range(2):
            pltpu.make_async_copy(
                out_v.at[b], out_hbm.at[pl.ds(0, chunk)], out_sems[b]).wait()

    return k


_KERNEL = _build(BATCH * FIELDS, 128, VOCAB, N_BLOCKS)


def kernel(x, qweight, absmax, code):
    n_rows = BATCH * FIELDS
    bpw = n_rows // NW
    xr = x.reshape(NW, bpw // 128, 128)
    out = _KERNEL(xr, qweight, absmax, code)
    return out.reshape(BATCH, FIELDS, DIM)


# speedup vs baseline: 455.1965x; 1.0048x over previous
import functools

import jax
import jax.numpy as jnp
from jax import lax
from jax.experimental import pallas as pl
from jax.experimental.pallas import tpu as pltpu
from jax.experimental.pallas import tpu_sc as plsc

VOCAB = 100000
DIM = 128
BLOCK = 4096
N_BLOCKS = (VOCAB * DIM) // BLOCK
ROWS_PER_ABSMAX = BLOCK // DIM
BATCH = 4096
FIELDS = 26

NC, NS, L = 2, 16, 16
NW = NC * NS


def _build(n_rows, chunk, vocab, n_blocks):
    bpw = n_rows // NW
    nchunk = bpw // chunk
    mesh = plsc.VectorSubcoreMesh(core_axis_name="c", subcore_axis_name="s")

    @functools.partial(
        pl.kernel,
        out_type=jax.ShapeDtypeStruct((n_rows, DIM), jnp.float32),
        mesh=mesh,
        compiler_params=pltpu.CompilerParams(
            use_tc_tiling_on_sc=False, needs_layout_passes=False),
        scratch_types=[
            pltpu.VMEM((nchunk, chunk), jnp.int32),
            pltpu.VMEM((n_blocks,), jnp.float32),
            pltpu.VMEM((256,), jnp.float32),
            pltpu.VMEM((2, chunk, DIM), jnp.int32),
            pltpu.VMEM((2, chunk, DIM), jnp.float32),
            pltpu.SemaphoreType.DMA,
            pltpu.SemaphoreType.DMA,
            pltpu.SemaphoreType.DMA,
            pltpu.SemaphoreType.DMA,
        ],
    )
    def k(x_hbm, qw_hbm, absmax_hbm, code_hbm, out_hbm,
          idx_v, absmax_v, code_v, rows_v, out_v,
          in_sem0, in_sem1, out_sem0, out_sem1):
        in_sems = [in_sem0, in_sem1]
        out_sems = [out_sem0, out_sem1]
        wid = lax.axis_index("s") * NC + lax.axis_index("c")
        pltpu.sync_copy(x_hbm.at[wid], idx_v)
        pltpu.sync_copy(absmax_hbm, absmax_v)
        pltpu.sync_copy(code_hbm, code_v)

        for b in range(2):
            pltpu.async_copy(qw_hbm.at[idx_v.at[b]], rows_v.at[b], in_sems[b])

        def dequant_chunk(c, b):
            def group_body(g, carry2):
                idx16 = idx_v[c, pl.ds(g * L, L)]
                shift5 = jnp.full((L,), 5, jnp.int32)
                s_g = plsc.load_gather(
                    absmax_v, [lax.shift_right_logical(idx16, shift5)])
                for j in range(L):
                    scale = jnp.take(s_g, jnp.full((L,), j, jnp.int32))
                    r = g * L + j
                    for kk in range(DIM // L):
                        q = rows_v[b, r, pl.ds(kk * L, L)]
                        out_v[b, r, pl.ds(kk * L, L)] = (
                            plsc.load_gather(code_v, [q]) * scale)
                return carry2

            lax.fori_loop(0, chunk // L, group_body, 0, unroll=False)

        def pair_body(g, carry):
            for b in range(2):
                c = g * 2 + b
                pltpu.make_async_copy(
                    qw_hbm.at[pl.ds(0, chunk)], rows_v.at[b],
                    in_sems[b]).wait()
                @pl.when(c >= 2)
                def _():
                    pltpu.make_async_copy(
                        out_v.at[b], out_hbm.at[pl.ds(0, chunk)],
                        out_sems[b]).wait()

                dequant_chunk(c, b)

                pltpu.async_copy(
                    out_v.at[b],
                    out_hbm.at[pl.ds(wid * bpw + c * chunk, chunk)],
                    out_sems[b])
                @pl.when(c + 2 < nchunk)
                def _():
                    pltpu.async_copy(
                        qw_hbm.at[idx_v.at[c + 2]], rows_v.at[b], in_sems[b])
            return carry

        lax.fori_loop(0, nchunk // 2, pair_body, 0, unroll=False)

        for b in range(2):
            pltpu.make_async_copy(
                out_v.at[b], out_hbm.at[pl.ds(0, chunk)], out_sems[b]).wait()

    return k


_KERNEL = _build(BATCH * FIELDS, 128, VOCAB, N_BLOCKS)


def kernel(x, qweight, absmax, code):
    n_rows = BATCH * FIELDS
    bpw = n_rows // NW
    xr = x.reshape(NW, bpw // 128, 128)
    out = _KERNEL(xr, qweight, absmax, code)
    return out.reshape(BATCH, FIELDS, DIM)


# speedup vs baseline: 587.9962x; 1.2917x over previous
import functools

import jax
import jax.numpy as jnp
from jax import lax
from jax.experimental import pallas as pl
from jax.experimental.pallas import tpu as pltpu
from jax.experimental.pallas import tpu_sc as plsc

VOCAB = 100000
DIM = 128
BLOCK = 4096
N_BLOCKS = (VOCAB * DIM) // BLOCK
ROWS_PER_ABSMAX = BLOCK // DIM
BATCH = 4096
FIELDS = 26

NC, NS, L = 2, 16, 16
NW = NC * NS


def _build(n_rows, chunk, vocab, n_blocks):
    bpw = n_rows // NW
    nchunk = bpw // chunk
    mesh = plsc.VectorSubcoreMesh(core_axis_name="c", subcore_axis_name="s")

    @functools.partial(
        pl.kernel,
        out_type=jax.ShapeDtypeStruct((n_rows, DIM), jnp.float32),
        mesh=mesh,
        compiler_params=pltpu.CompilerParams(
            use_tc_tiling_on_sc=False, needs_layout_passes=False),
        scratch_types=[
            pltpu.VMEM((nchunk, chunk), jnp.int32),
            pltpu.VMEM((n_blocks,), jnp.float32),
            pltpu.VMEM((256,), jnp.float32),
            pltpu.VMEM((2, chunk, DIM), jnp.int32),
            pltpu.VMEM((2, chunk, DIM), jnp.float32),
            pltpu.SemaphoreType.DMA,
            pltpu.SemaphoreType.DMA,
            pltpu.SemaphoreType.DMA,
            pltpu.SemaphoreType.DMA,
        ],
    )
    def k(x_hbm, qw_hbm, absmax_hbm, code_hbm, out_hbm,
          idx_v, absmax_v, code_v, rows_v, out_v,
          in_sem0, in_sem1, out_sem0, out_sem1):
        in_sems = [in_sem0, in_sem1]
        out_sems = [out_sem0, out_sem1]
        wid = lax.axis_index("s") * NC + lax.axis_index("c")
        pltpu.sync_copy(x_hbm.at[wid], idx_v)
        pltpu.sync_copy(absmax_hbm, absmax_v)
        pltpu.sync_copy(code_hbm, code_v)

        for b in range(2):
            pltpu.async_copy(qw_hbm.at[idx_v.at[b]], rows_v.at[b], in_sems[b])

        def dequant_chunk(c, b):
            @plsc.parallel_loop(0, chunk // L)
            def group_body(g):
                idx16 = idx_v[c, pl.ds(g * L, L)]
                shift5 = jnp.full((L,), 5, jnp.int32)
                s_g = plsc.load_gather(
                    absmax_v, [lax.shift_right_logical(idx16, shift5)])
                for j in range(L):
                    scale = jnp.take(s_g, jnp.full((L,), j, jnp.int32))
                    r = g * L + j
                    for kk in range(DIM // L):
                        q = rows_v[b, r, pl.ds(kk * L, L)]
                        out_v[b, r, pl.ds(kk * L, L)] = (
                            plsc.load_gather(code_v, [q]) * scale)

        def pair_body(g, carry):
            for b in range(2):
                c = g * 2 + b
                pltpu.make_async_copy(
                    qw_hbm.at[pl.ds(0, chunk)], rows_v.at[b],
                    in_sems[b]).wait()
                @pl.when(c >= 2)
                def _():
                    pltpu.make_async_copy(
                        out_v.at[b], out_hbm.at[pl.ds(0, chunk)],
                        out_sems[b]).wait()

                dequant_chunk(c, b)

                pltpu.async_copy(
                    out_v.at[b],
                    out_hbm.at[pl.ds(wid * bpw + c * chunk, chunk)],
                    out_sems[b])
                @pl.when(c + 2 < nchunk)
                def _():
                    pltpu.async_copy(
                        qw_hbm.at[idx_v.at[c + 2]], rows_v.at[b], in_sems[b])
            return carry

        lax.fori_loop(0, nchunk // 2, pair_body, 0, unroll=False)

        for b in range(2):
            pltpu.make_async_copy(
                out_v.at[b], out_hbm.at[pl.ds(0, chunk)], out_sems[b]).wait()

    return k


_KERNEL = _build(BATCH * FIELDS, 128, VOCAB, N_BLOCKS)


def kernel(x, qweight, absmax, code):
    n_rows = BATCH * FIELDS
    bpw = n_rows // NW
    xr = x.reshape(NW, bpw // 128, 128)
    out = _KERNEL(xr, qweight, absmax, code)
    return out.reshape(BATCH, FIELDS, DIM)


# speedup vs baseline: 641.8322x; 1.0916x over previous
import functools

import jax
import jax.numpy as jnp
from jax import lax
from jax.experimental import pallas as pl
from jax.experimental.pallas import tpu as pltpu
from jax.experimental.pallas import tpu_sc as plsc

VOCAB = 100000
DIM = 128
BLOCK = 4096
N_BLOCKS = (VOCAB * DIM) // BLOCK
ROWS_PER_ABSMAX = BLOCK // DIM
BATCH = 4096
FIELDS = 26

NC, NS, L = 2, 16, 16
NW = NC * NS


def _build(n_rows, chunk, vocab, n_blocks):
    bpw = n_rows // NW
    nchunk = bpw // chunk
    mesh = plsc.VectorSubcoreMesh(core_axis_name="c", subcore_axis_name="s")

    @functools.partial(
        pl.kernel,
        out_type=jax.ShapeDtypeStruct((n_rows, DIM), jnp.float32),
        mesh=mesh,
        compiler_params=pltpu.CompilerParams(
            use_tc_tiling_on_sc=False, needs_layout_passes=False),
        scratch_types=[
            pltpu.VMEM((nchunk, chunk), jnp.int32),
            pltpu.VMEM((n_blocks,), jnp.float32),
            pltpu.VMEM((256,), jnp.float32),
            pltpu.VMEM((2, chunk, DIM), jnp.int32),
            pltpu.VMEM((2, chunk, DIM), jnp.float32),
            pltpu.SemaphoreType.DMA,
            pltpu.SemaphoreType.DMA,
            pltpu.SemaphoreType.DMA,
            pltpu.SemaphoreType.DMA,
        ],
    )
    def k(x_hbm, qw_hbm, absmax_hbm, code_hbm, out_hbm,
          idx_v, absmax_v, code_v, rows_v, out_v,
          in_sem0, in_sem1, out_sem0, out_sem1):
        in_sems = [in_sem0, in_sem1]
        out_sems = [out_sem0, out_sem1]
        wid = lax.axis_index("s") * NC + lax.axis_index("c")
        pltpu.sync_copy(x_hbm.at[wid], idx_v)
        pltpu.sync_copy(absmax_hbm, absmax_v)
        pltpu.sync_copy(code_hbm, code_v)

        for b in range(2):
            pltpu.async_copy(qw_hbm.at[idx_v.at[b]], rows_v.at[b], in_sems[b])

        def dequant_chunk(c, b):
            @plsc.parallel_loop(0, chunk // L)
            def group_body(g):
                idx16 = idx_v[c, pl.ds(g * L, L)]
                shift5 = jnp.full((L,), 5, jnp.int32)
                s_g = plsc.load_gather(
                    absmax_v, [lax.shift_right_logical(idx16, shift5)])
                for j in range(L):
                    scale = jnp.take(s_g, jnp.full((L,), j, jnp.int32))
                    r = g * L + j
                    for kk in range(DIM // L):
                        q = rows_v[b, r, pl.ds(kk * L, L)]
                        out_v[b, r, pl.ds(kk * L, L)] = (
                            plsc.load_gather(code_v, [q]) * scale)

        def pair_body(g, carry):
            for b in range(2):
                c = g * 2 + b
                pltpu.make_async_copy(
                    qw_hbm.at[pl.ds(0, chunk)], rows_v.at[b],
                    in_sems[b]).wait()
                @pl.when(c >= 2)
                def _():
                    pltpu.make_async_copy(
                        out_v.at[b], out_hbm.at[pl.ds(0, chunk)],
                        out_sems[b]).wait()

                dequant_chunk(c, b)

                pltpu.async_copy(
                    out_v.at[b],
                    out_hbm.at[pl.ds(wid * bpw + c * chunk, chunk)],
                    out_sems[b])
                @pl.when(c + 2 < nchunk)
                def _():
                    pltpu.async_copy(
                        qw_hbm.at[idx_v.at[c + 2]], rows_v.at[b], in_sems[b])
            return carry

        lax.fori_loop(0, nchunk // 2, pair_body, 0, unroll=False)

        for b in range(2):
            pltpu.make_async_copy(
                out_v.at[b], out_hbm.at[pl.ds(0, chunk)], out_sems[b]).wait()

    return k


_CHUNK = 208
_KERNEL = _build(BATCH * FIELDS, _CHUNK, VOCAB, N_BLOCKS)


def kernel(x, qweight, absmax, code):
    n_rows = BATCH * FIELDS
    bpw = n_rows // NW
    xr = x.reshape(NW, bpw // _CHUNK, _CHUNK)
    out = _KERNEL(xr, qweight, absmax, code)
    return out.reshape(BATCH, FIELDS, DIM)
